# Initial kernel scaffold; baseline (speedup 1.0000x reference)
import jax
import jax.numpy as jnp
from jax.experimental import pallas as pl


def kernel(x, edge_index, batch, W1a, b1a, W1b, b1b, W2a, b2a, W2b, b2b):
    raise NotImplementedError("write your pallas kernel here")


# speedup vs baseline: 4.1394x; 4.1394x over previous
import functools

import jax
import jax.numpy as jnp
from jax import lax
from jax.experimental import pallas as pl
from jax.experimental.pallas import tpu as pltpu
from jax.experimental.pallas import tpu_sc as plsc

_N = 10000
_E = 320000
_D = 128
_G = 64

_NC = 2
_NS = 16
_NW = _NC * _NS
_CHUNK = 128
_CT = -(-_E // (_NW * _CHUNK))
_EPAD = _NW * _CT * _CHUNK
_NACC = 10112
_ZROWS = _NACC // _NS
_OROWS = _NACC // _NS

_BN = 1000
_NBLK = _N // _BN


@functools.cache
def _make_sc_agg():
    mesh = plsc.VectorSubcoreMesh(core_axis_name="c", subcore_axis_name="s")

    @functools.partial(
        pl.kernel,
        mesh=mesh,
        out_type=jax.ShapeDtypeStruct((_NC, _NACC, _D), jnp.float32),
        scratch_types=[
            pltpu.VMEM((_CT, _CHUNK), jnp.int32),
            pltpu.VMEM((_CT, _CHUNK), jnp.int32),
            pltpu.VMEM((_CHUNK, _D), jnp.float32),
            pltpu.VMEM_SHARED((_NACC, _D), jnp.float32),
            pltpu.SemaphoreType.DMA,
        ],
    )
    def agg(feat_hbm, src_hbm, dst_hbm, zeros_hbm, out_hbm,
            src_v, dst_v, rows_v, acc_sh, sem):
        c = lax.axis_index("c")
        s = lax.axis_index("s")
        wid = c * _NS + s

        pltpu.sync_copy(zeros_hbm, acc_sh.at[pl.ds(s * _ZROWS, _ZROWS)])
        pltpu.sync_copy(src_hbm.at[wid], src_v)
        pltpu.sync_copy(dst_hbm.at[wid], dst_v)
        plsc.subcore_barrier()

        def body(j, carry):
            pltpu.async_copy(feat_hbm.at[src_v.at[j]], rows_v, sem).wait()
            pltpu.sync_copy(rows_v, acc_sh.at[dst_v.at[j]], add=True)
            return carry

        lax.fori_loop(0, _CT, body, 0, unroll=False)

        plsc.subcore_barrier()
        pltpu.sync_copy(acc_sh.at[pl.ds(s * _OROWS, _OROWS)],
                        out_hbm.at[c, pl.ds(s * _OROWS, _OROWS)])

    return agg


def _sc_agg(feat, srcp, dstp, zrows):
    return _make_sc_agg()(feat, srcp, dstp, zrows)


def _mlp_body(x_ref, agg_ref, wa_ref, ba_ref, wb_ref, bb_ref, out_ref):
    h = x_ref[...] + agg_ref[0] + agg_ref[1]
    h = jnp.dot(h, wa_ref[...], preferred_element_type=jnp.float32) + ba_ref[...]
    h = jnp.maximum(h, 0.0)
    h = jnp.dot(h, wb_ref[...], preferred_element_type=jnp.float32) + bb_ref[...]
    out_ref[...] = jnp.maximum(h, 0.0)


def _tc_mlp1(x, agg, wa, ba, wb, bb):
    blk = lambda i: (i, 0)
    full = lambda i: (0, 0)
    return pl.pallas_call(
        _mlp_body,
        grid=(_NBLK,),
        in_specs=[
            pl.BlockSpec((_BN, _D), blk),
            pl.BlockSpec((_NC, _BN, _D), lambda i: (0, i, 0)),
            pl.BlockSpec((_D, _D), full),
            pl.BlockSpec((1, _D), full),
            pl.BlockSpec((_D, _D), full),
            pl.BlockSpec((1, _D), full),
        ],
        out_specs=pl.BlockSpec((_BN, _D), blk),
        out_shape=jax.ShapeDtypeStruct((_N, _D), jnp.float32),
    )(x, agg, wa, ba, wb, bb)


def _mlp_pool_body(batch_ref, x_ref, agg_ref, wa_ref, ba_ref, wb_ref,
                   bb_ref, out_ref, pool_ref):
    i = pl.program_id(0)
    h = x_ref[...] + agg_ref[0] + agg_ref[1]
    h = jnp.dot(h, wa_ref[...], preferred_element_type=jnp.float32) + ba_ref[...]
    h = jnp.maximum(h, 0.0)
    h = jnp.dot(h, wb_ref[...], preferred_element_type=jnp.float32) + bb_ref[...]
    out_ref[...] = h
    b = batch_ref[0, 0, :]
    onehot = (b[:, None] == lax.broadcasted_iota(jnp.int32, (_BN, _G), 1))
    contrib = lax.dot_general(onehot.astype(jnp.float32), h,
                              (((0,), (0,)), ((), ())),
                              preferred_element_type=jnp.float32)

    @pl.when(i == 0)
    def _init():
        pool_ref[...] = jnp.zeros_like(pool_ref)

    pool_ref[...] += contrib


def _tc_mlp2_pool(batch3, x, agg, wa, ba, wb, bb):
    blk = lambda i: (i, 0)
    full = lambda i: (0, 0)
    return pl.pallas_call(
        _mlp_pool_body,
        grid=(_NBLK,),
        in_specs=[
            pl.BlockSpec((1, 1, _BN), lambda i: (i, 0, 0)),
            pl.BlockSpec((_BN, _D), blk),
            pl.BlockSpec((_NC, _BN, _D), lambda i: (0, i, 0)),
            pl.BlockSpec((_D, _D), full),
            pl.BlockSpec((1, _D), full),
            pl.BlockSpec((_D, _D), full),
            pl.BlockSpec((1, _D), full),
        ],
        out_specs=[
            pl.BlockSpec((_BN, _D), blk),
            pl.BlockSpec((_G, _D), full),
        ],
        out_shape=[
            jax.ShapeDtypeStruct((_N, _D), jnp.float32),
            jax.ShapeDtypeStruct((_G, _D), jnp.float32),
        ],
    )(batch3, x, agg, wa, ba, wb, bb)


def kernel(x, edge_index, batch, W1a, b1a, W1b, b1b, W2a, b2a, W2b, b2b):
    src = edge_index[0]
    dst = edge_index[1]
    pad = _EPAD - _E
    srcp = jnp.concatenate([src, jnp.zeros((pad,), jnp.int32)])
    dstp = jnp.concatenate([dst, jnp.full((pad,), _N, jnp.int32)])
    srcp = srcp.reshape(_NW, _CT, _CHUNK)
    dstp = dstp.reshape(_NW, _CT, _CHUNK)
    zrows = jnp.zeros((_ZROWS, _D), jnp.float32)

    ba1 = b1a.reshape(1, _D)
    bb1 = b1b.reshape(1, _D)
    ba2 = b2a.reshape(1, _D)
    bb2 = b2b.reshape(1, _D)

    agg1 = _sc_agg(x, srcp, dstp, zrows)
    h1 = _tc_mlp1(x, agg1, W1a, ba1, W1b, bb1)
    agg2 = _sc_agg(h1, srcp, dstp, zrows)
    batch3 = batch.reshape(_NBLK, 1, _BN)
    h2, pooled = _tc_mlp2_pool(batch3, h1, agg2, W2a, ba2, W2b, bb2)
    return (pooled, h2)
